# Initial kernel scaffold; baseline (speedup 1.0000x reference)
import jax
import jax.numpy as jnp
from jax.experimental import pallas as pl


def kernel(x, edge_index, W1, b1, W2, b2, Wfc, bfc):
    raise NotImplementedError("write your pallas kernel here")


# speedup vs baseline: 8.8861x; 8.8861x over previous
import functools

import jax
import jax.numpy as jnp
from jax import lax
from jax.experimental import pallas as pl
from jax.experimental.pallas import tpu as pltpu
from jax.experimental.pallas import tpu_sc as plsc

_N = 10000
_E = 160000
_D = 256
_H1 = 512
_H2 = 128
_C = 4

_L = 128
_NACC = 10240
_RPT = _NACC // 16
_SENT = _N
_EP = 163840
_NB1 = _EP // (16 * _L)
_NB2 = _EP // (32 * _L)


def _zero_rows(buf, ncols):
    z = jnp.zeros((16,), jnp.float32)

    def body(r, _):
        for k in range(ncols // 16):
            buf[r, pl.ds(k * 16, 16)] = z
        return 0

    lax.fori_loop(0, 128, body, 0)


def _fill_rows(buf, ncols, val):
    v = jnp.full((16,), val, jnp.float32)

    def body(r, _):
        for k in range(ncols // 16):
            buf[r, pl.ds(k * 16, 16)] = v
        return 0

    lax.fori_loop(0, 128, body, 0)


def _make_deg_kernel():
    mesh = plsc.VectorSubcoreMesh(core_axis_name="c", subcore_axis_name="s")

    @functools.partial(
        pl.kernel,
        out_type=jax.ShapeDtypeStruct((2 * _NACC, 128), jnp.float32),
        mesh=mesh,
        scratch_types=[
            pltpu.VMEM((_NB2, _L), jnp.int32),
            pltpu.VMEM((128, 128), jnp.float32),
            pltpu.VMEM_SHARED((_NACC, 128), jnp.float32),
        ],
    )
    def deg_kernel(dst_hbm, out_hbm, dst_v, gbuf, acc):
        c = lax.axis_index("c")
        s = lax.axis_index("s")
        w = c * 16 + s
        _zero_rows(gbuf, 128)
        base = s * _RPT
        for k in range(_RPT // 128):
            pltpu.sync_copy(gbuf, acc.at[pl.ds(base + k * 128, 128)])
        plsc.subcore_barrier()
        _fill_rows(gbuf, 128, 1.0)
        pltpu.sync_copy(dst_hbm.at[w], dst_v)

        def body(j, _):
            pltpu.sync_copy(gbuf, acc.at[dst_v.at[j]], add=True)
            return 0

        lax.fori_loop(0, _NB2, body, 0)
        plsc.subcore_barrier()
        pltpu.sync_copy(
            acc.at[pl.ds(base, _RPT)],
            out_hbm.at[pl.ds(c * _NACC + base, _RPT)],
        )

    return deg_kernel


def _make_scatter_kernel(nb):
    mesh = plsc.VectorSubcoreMesh(core_axis_name="c", subcore_axis_name="s")

    def body_fn(table_hbm, src_hbm, dst_hbm, out_hbm, src_v, dst_v, gbuf,
                acc):
        c = lax.axis_index("c")
        s = lax.axis_index("s")
        w = c * 16 + s
        _zero_rows(gbuf, 128)
        base = s * _RPT
        for k in range(_RPT // 128):
            pltpu.sync_copy(gbuf, acc.at[pl.ds(base + k * 128, 128)])
        plsc.subcore_barrier()
        pltpu.sync_copy(src_hbm.at[w], src_v)
        pltpu.sync_copy(dst_hbm.at[w], dst_v)

        def body(j, _):
            pltpu.sync_copy(table_hbm.at[src_v.at[j]], gbuf)
            pltpu.sync_copy(gbuf, acc.at[dst_v.at[j]], add=True)
            return 0

        lax.fori_loop(0, nb, body, 0)
        plsc.subcore_barrier()
        pltpu.sync_copy(
            acc.at[pl.ds(base, _RPT)],
            out_hbm.at[pl.ds(c * _NACC + base, _RPT)],
        )

    return functools.partial(
        pl.kernel,
        out_type=jax.ShapeDtypeStruct((2 * _NACC, 128), jnp.float32),
        mesh=mesh,
        scratch_types=[
            pltpu.VMEM((nb, _L), jnp.int32),
            pltpu.VMEM((nb, _L), jnp.int32),
            pltpu.VMEM((_L, 128), jnp.float32),
            pltpu.VMEM_SHARED((_NACC, 128), jnp.float32),
        ],
    )(body_fn)


_BN = 1000


def _tc_prep(x, d0, d1):
    def body(x_ref, d0_ref, d1_ref, g_ref, dis_ref):
        deg = d0_ref[:, :1] + d1_ref[:, :1] + 1.0
        dis = lax.rsqrt(deg)
        dis_ref[...] = dis
        g_ref[...] = x_ref[...] * dis

    nblk = _N // _BN
    return pl.pallas_call(
        body,
        grid=(2, nblk),
        in_specs=[
            pl.BlockSpec((_BN, 128), lambda h, i: (i, h)),
            pl.BlockSpec((_BN, 128), lambda h, i: (i, 0)),
            pl.BlockSpec((_BN, 128), lambda h, i: (i, 0)),
        ],
        out_specs=[
            pl.BlockSpec((_BN, 128), lambda h, i: (h * nblk + i, 0)),
            pl.BlockSpec((_BN, 1), lambda h, i: (i, 0)),
        ],
        out_shape=[
            jax.ShapeDtypeStruct((2 * _N, 128), jnp.float32),
            jax.ShapeDtypeStruct((_N, 1), jnp.float32),
        ],
    )(x, d0, d1)


def _tc_mid(sa, sb, ga, gb, dis, w1a, w1b, b1, w2):
    def body(sa_ref, sb_ref, ga_ref, gb_ref, dis_ref, w1a_ref, w1b_ref,
             b1_ref, w2_ref, g2_ref):
        dis = dis_ref[...]
        aa = (sa_ref[...] + ga_ref[...]) * dis
        ab = (sb_ref[...] + gb_ref[...]) * dis
        h1 = jnp.dot(aa, w1a_ref[...], preferred_element_type=jnp.float32)
        h1 += jnp.dot(ab, w1b_ref[...], preferred_element_type=jnp.float32)
        h1 = jnp.maximum(h1 + b1_ref[...], 0.0)
        p2 = jnp.dot(h1, w2_ref[...], preferred_element_type=jnp.float32)
        g2_ref[...] = p2 * dis

    nblk = _N // _BN
    return pl.pallas_call(
        body,
        grid=(nblk,),
        in_specs=[
            pl.BlockSpec((_BN, 128), lambda i: (i, 0)),
            pl.BlockSpec((_BN, 128), lambda i: (i, 0)),
            pl.BlockSpec((_BN, 128), lambda i: (i, 0)),
            pl.BlockSpec((_BN, 128), lambda i: (i, 0)),
            pl.BlockSpec((_BN, 1), lambda i: (i, 0)),
            pl.BlockSpec((128, _H1), lambda i: (0, 0)),
            pl.BlockSpec((128, _H1), lambda i: (0, 0)),
            pl.BlockSpec((1, _H1), lambda i: (0, 0)),
            pl.BlockSpec((_H1, _H2), lambda i: (0, 0)),
        ],
        out_specs=pl.BlockSpec((_BN, _H2), lambda i: (i, 0)),
        out_shape=jax.ShapeDtypeStruct((_N, _H2), jnp.float32),
    )(sa, sb, ga, gb, dis, w1a, w1b, b1, w2)


def _tc_final(s2a, s2b, g2, dis, b2, wfc, bfc):
    def body(sa_ref, sb_ref, g2_ref, dis_ref, b2_ref, wfc_ref, bfc_ref,
             out_ref):
        a = (sa_ref[...] + sb_ref[...] + g2_ref[...]) * dis_ref[...]
        h2 = jnp.maximum(a + b2_ref[...], 0.0)
        logits = jnp.dot(h2, wfc_ref[...], preferred_element_type=jnp.float32)
        logits += bfc_ref[...]
        m = jnp.max(logits, axis=1, keepdims=True)
        z = logits - m
        lse = jnp.log(jnp.sum(jnp.exp(z), axis=1, keepdims=True))
        out_ref[...] = z - lse

    nblk = _N // _BN
    return pl.pallas_call(
        body,
        grid=(nblk,),
        in_specs=[
            pl.BlockSpec((_BN, _H2), lambda i: (i, 0)),
            pl.BlockSpec((_BN, _H2), lambda i: (i, 0)),
            pl.BlockSpec((_BN, _H2), lambda i: (i, 0)),
            pl.BlockSpec((_BN, 1), lambda i: (i, 0)),
            pl.BlockSpec((1, _H2), lambda i: (0, 0)),
            pl.BlockSpec((_H2, _C), lambda i: (0, 0)),
            pl.BlockSpec((1, _C), lambda i: (0, 0)),
        ],
        out_specs=pl.BlockSpec((_BN, _C), lambda i: (i, 0)),
        out_shape=jax.ShapeDtypeStruct((_N, _C), jnp.float32),
    )(s2a, s2b, g2, dis, b2, wfc, bfc)


@jax.jit
def kernel(x, edge_index, W1, b1, W2, b2, Wfc, bfc):
    src = edge_index[0]
    dst = edge_index[1]
    pad = _EP - _E
    srcp = jnp.concatenate([src, jnp.zeros((pad,), jnp.int32)])
    dstp = jnp.concatenate([dst, jnp.full((pad,), _SENT, jnp.int32)])

    src2 = srcp.reshape(32, _NB2, _L)
    dst2 = dstp.reshape(32, _NB2, _L)
    src1 = (srcp.reshape(1, 16, _NB1, _L)
            + jnp.array([0, _N], jnp.int32).reshape(2, 1, 1, 1)
            ).reshape(32, _NB1, _L)
    dst1 = jnp.broadcast_to(
        dstp.reshape(1, 16, _NB1, _L), (2, 16, _NB1, _L)
    ).reshape(32, _NB1, _L)

    degcat = _make_deg_kernel()(dst2)
    d0 = degcat[:_N]
    d1 = degcat[_NACC:_NACC + _N]

    gcat, dis = _tc_prep(x, d0, d1)

    s1cat = _make_scatter_kernel(_NB1)(gcat, src1, dst1)
    g2 = _tc_mid(
        s1cat[:_N], s1cat[_NACC:_NACC + _N],
        gcat[:_N], gcat[_N:],
        dis, W1[:128], W1[128:], b1.reshape(1, _H1), W2,
    )

    s2cat = _make_scatter_kernel(_NB2)(g2, src2, dst2)
    return _tc_final(
        s2cat[:_N], s2cat[_NACC:_NACC + _N], g2, dis,
        b2.reshape(1, _H2), Wfc, bfc.reshape(1, _C),
    )


# speedup vs baseline: 9.6509x; 1.0861x over previous
import functools

import jax
import jax.numpy as jnp
from jax import lax
from jax.experimental import pallas as pl
from jax.experimental.pallas import tpu as pltpu
from jax.experimental.pallas import tpu_sc as plsc

_N = 10000
_E = 160000
_D = 256
_H1 = 512
_H2 = 128
_C = 4

_L = 128
_NACC = 10240
_RPT = _NACC // 16
_SENT = _N
_EP = 163840
_NB1 = _EP // (16 * _L)
_NB2 = _EP // (32 * _L)


def _zero_rows(buf, ncols):
    z = jnp.zeros((16,), jnp.float32)

    def body(r, _):
        for k in range(ncols // 16):
            buf[r, pl.ds(k * 16, 16)] = z
        return 0

    lax.fori_loop(0, 128, body, 0)


def _fill_rows(buf, ncols, val):
    v = jnp.full((16,), val, jnp.float32)

    def body(r, _):
        for k in range(ncols // 16):
            buf[r, pl.ds(k * 16, 16)] = v
        return 0

    lax.fori_loop(0, 128, body, 0)


def _make_deg_kernel():
    mesh = plsc.VectorSubcoreMesh(core_axis_name="c", subcore_axis_name="s")

    @functools.partial(
        pl.kernel,
        out_type=jax.ShapeDtypeStruct((2 * _NACC, 128), jnp.float32),
        mesh=mesh,
        scratch_types=[
            pltpu.VMEM((_NB2, _L), jnp.int32),
            pltpu.VMEM((128, 128), jnp.float32),
            pltpu.VMEM_SHARED((_NACC, 128), jnp.float32),
        ],
    )
    def deg_kernel(dst_hbm, out_hbm, dst_v, gbuf, acc):
        c = lax.axis_index("c")
        s = lax.axis_index("s")
        w = c * 16 + s
        _zero_rows(gbuf, 128)
        base = s * _RPT
        for k in range(_RPT // 128):
            pltpu.sync_copy(gbuf, acc.at[pl.ds(base + k * 128, 128)])
        plsc.subcore_barrier()
        _fill_rows(gbuf, 128, 1.0)
        pltpu.sync_copy(dst_hbm.at[w], dst_v)

        def body(j, _):
            pltpu.sync_copy(gbuf, acc.at[dst_v.at[j]], add=True)
            return 0

        lax.fori_loop(0, _NB2, body, 0)
        plsc.subcore_barrier()
        pltpu.sync_copy(
            acc.at[pl.ds(base, _RPT)],
            out_hbm.at[pl.ds(c * _NACC + base, _RPT)],
        )

    return deg_kernel


def _make_scatter_kernel(nb):
    mesh = plsc.VectorSubcoreMesh(core_axis_name="c", subcore_axis_name="s")

    stage = 40
    nstg = nb // stage

    def body_fn(table_hbm, src_hbm, dst_hbm, out_hbm, src_v, dst_v, buf_a,
                buf_b, sem_a, sem_b, acc):
        c = lax.axis_index("c")
        s = lax.axis_index("s")
        w = c * 16 + s
        _zero_rows(buf_a, 128)
        base = s * _RPT
        for k in range(_RPT // 128):
            pltpu.sync_copy(buf_a, acc.at[pl.ds(base + k * 128, 128)])
        plsc.subcore_barrier()

        def wait(buf, sem):
            pltpu.make_async_copy(table_hbm.at[pl.ds(0, _L)], buf, sem).wait()

        def run_stage(st, _):
            pltpu.sync_copy(src_hbm.at[w, pl.ds(st * stage, stage)], src_v)
            pltpu.sync_copy(dst_hbm.at[w, pl.ds(st * stage, stage)], dst_v)
            pltpu.async_copy(table_hbm.at[src_v.at[0]], buf_a, sem_a)

            def pipe(m, _):
                j0 = 2 * m
                wait(buf_a, sem_a)
                pltpu.async_copy(table_hbm.at[src_v.at[j0 + 1]], buf_b, sem_b)
                pltpu.sync_copy(buf_a, acc.at[dst_v.at[j0]], add=True)
                wait(buf_b, sem_b)
                pltpu.async_copy(table_hbm.at[src_v.at[j0 + 2]], buf_a, sem_a)
                pltpu.sync_copy(buf_b, acc.at[dst_v.at[j0 + 1]], add=True)
                return 0

            lax.fori_loop(0, (stage - 2) // 2, pipe, 0)
            wait(buf_a, sem_a)
            pltpu.async_copy(table_hbm.at[src_v.at[stage - 1]], buf_b, sem_b)
            pltpu.sync_copy(buf_a, acc.at[dst_v.at[stage - 2]], add=True)
            wait(buf_b, sem_b)
            pltpu.sync_copy(buf_b, acc.at[dst_v.at[stage - 1]], add=True)
            return 0

        lax.fori_loop(0, nstg, run_stage, 0)
        plsc.subcore_barrier()
        pltpu.sync_copy(
            acc.at[pl.ds(base, _RPT)],
            out_hbm.at[pl.ds(c * _NACC + base, _RPT)],
        )

    return functools.partial(
        pl.kernel,
        out_type=jax.ShapeDtypeStruct((2 * _NACC, 128), jnp.float32),
        mesh=mesh,
        scratch_types=[
            pltpu.VMEM((stage, _L), jnp.int32),
            pltpu.VMEM((stage, _L), jnp.int32),
            pltpu.VMEM((_L, 128), jnp.float32),
            pltpu.VMEM((_L, 128), jnp.float32),
            pltpu.SemaphoreType.DMA,
            pltpu.SemaphoreType.DMA,
            pltpu.VMEM_SHARED((_NACC, 128), jnp.float32),
        ],
    )(body_fn)


_BN = 1000


def _tc_prep(x, d0, d1):
    def body(x_ref, d0_ref, d1_ref, g_ref, dis_ref):
        deg = d0_ref[:, :1] + d1_ref[:, :1] + 1.0
        dis = lax.rsqrt(deg)
        dis_ref[...] = dis
        g_ref[...] = x_ref[...] * dis

    nblk = _N // _BN
    return pl.pallas_call(
        body,
        grid=(2, nblk),
        in_specs=[
            pl.BlockSpec((_BN, 128), lambda h, i: (i, h)),
            pl.BlockSpec((_BN, 128), lambda h, i: (i, 0)),
            pl.BlockSpec((_BN, 128), lambda h, i: (i, 0)),
        ],
        out_specs=[
            pl.BlockSpec((_BN, 128), lambda h, i: (h * nblk + i, 0)),
            pl.BlockSpec((_BN, 1), lambda h, i: (i, 0)),
        ],
        out_shape=[
            jax.ShapeDtypeStruct((2 * _N, 128), jnp.float32),
            jax.ShapeDtypeStruct((_N, 1), jnp.float32),
        ],
    )(x, d0, d1)


def _tc_mid(sa, sb, ga, gb, dis, w1a, w1b, b1, w2):
    def body(sa_ref, sb_ref, ga_ref, gb_ref, dis_ref, w1a_ref, w1b_ref,
             b1_ref, w2_ref, g2_ref):
        dis = dis_ref[...]
        aa = (sa_ref[...] + ga_ref[...]) * dis
        ab = (sb_ref[...] + gb_ref[...]) * dis
        h1 = jnp.dot(aa, w1a_ref[...], preferred_element_type=jnp.float32)
        h1 += jnp.dot(ab, w1b_ref[...], preferred_element_type=jnp.float32)
        h1 = jnp.maximum(h1 + b1_ref[...], 0.0)
        p2 = jnp.dot(h1, w2_ref[...], preferred_element_type=jnp.float32)
        g2_ref[...] = p2 * dis

    nblk = _N // _BN
    return pl.pallas_call(
        body,
        grid=(nblk,),
        in_specs=[
            pl.BlockSpec((_BN, 128), lambda i: (i, 0)),
            pl.BlockSpec((_BN, 128), lambda i: (i, 0)),
            pl.BlockSpec((_BN, 128), lambda i: (i, 0)),
            pl.BlockSpec((_BN, 128), lambda i: (i, 0)),
            pl.BlockSpec((_BN, 1), lambda i: (i, 0)),
            pl.BlockSpec((128, _H1), lambda i: (0, 0)),
            pl.BlockSpec((128, _H1), lambda i: (0, 0)),
            pl.BlockSpec((1, _H1), lambda i: (0, 0)),
            pl.BlockSpec((_H1, _H2), lambda i: (0, 0)),
        ],
        out_specs=pl.BlockSpec((_BN, _H2), lambda i: (i, 0)),
        out_shape=jax.ShapeDtypeStruct((_N, _H2), jnp.float32),
    )(sa, sb, ga, gb, dis, w1a, w1b, b1, w2)


def _tc_final(s2a, s2b, g2, dis, b2, wfc, bfc):
    def body(sa_ref, sb_ref, g2_ref, dis_ref, b2_ref, wfc_ref, bfc_ref,
             out_ref):
        a = (sa_ref[...] + sb_ref[...] + g2_ref[...]) * dis_ref[...]
        h2 = jnp.maximum(a + b2_ref[...], 0.0)
        logits = jnp.dot(h2, wfc_ref[...], preferred_element_type=jnp.float32)
        logits += bfc_ref[...]
        m = jnp.max(logits, axis=1, keepdims=True)
        z = logits - m
        lse = jnp.log(jnp.sum(jnp.exp(z), axis=1, keepdims=True))
        out_ref[...] = z - lse

    nblk = _N // _BN
    return pl.pallas_call(
        body,
        grid=(nblk,),
        in_specs=[
            pl.BlockSpec((_BN, _H2), lambda i: (i, 0)),
            pl.BlockSpec((_BN, _H2), lambda i: (i, 0)),
            pl.BlockSpec((_BN, _H2), lambda i: (i, 0)),
            pl.BlockSpec((_BN, 1), lambda i: (i, 0)),
            pl.BlockSpec((1, _H2), lambda i: (0, 0)),
            pl.BlockSpec((_H2, _C), lambda i: (0, 0)),
            pl.BlockSpec((1, _C), lambda i: (0, 0)),
        ],
        out_specs=pl.BlockSpec((_BN, _C), lambda i: (i, 0)),
        out_shape=jax.ShapeDtypeStruct((_N, _C), jnp.float32),
    )(s2a, s2b, g2, dis, b2, wfc, bfc)


@jax.jit
def kernel(x, edge_index, W1, b1, W2, b2, Wfc, bfc):
    src = edge_index[0]
    dst = edge_index[1]
    pad = _EP - _E
    srcp = jnp.concatenate([src, jnp.zeros((pad,), jnp.int32)])
    dstp = jnp.concatenate([dst, jnp.full((pad,), _SENT, jnp.int32)])

    src2 = srcp.reshape(32, _NB2, _L)
    dst2 = dstp.reshape(32, _NB2, _L)
    src1 = (srcp.reshape(1, 16, _NB1, _L)
            + jnp.array([0, _N], jnp.int32).reshape(2, 1, 1, 1)
            ).reshape(32, _NB1, _L)
    dst1 = jnp.broadcast_to(
        dstp.reshape(1, 16, _NB1, _L), (2, 16, _NB1, _L)
    ).reshape(32, _NB1, _L)

    degcat = _make_deg_kernel()(dst2)
    d0 = degcat[:_N]
    d1 = degcat[_NACC:_NACC + _N]

    gcat, dis = _tc_prep(x, d0, d1)

    s1cat = _make_scatter_kernel(_NB1)(gcat, src1, dst1)
    g2 = _tc_mid(
        s1cat[:_N], s1cat[_NACC:_NACC + _N],
        gcat[:_N], gcat[_N:],
        dis, W1[:128], W1[128:], b1.reshape(1, _H1), W2,
    )

    s2cat = _make_scatter_kernel(_NB2)(g2, src2, dst2)
    return _tc_final(
        s2cat[:_N], s2cat[_NACC:_NACC + _N], g2, dis,
        b2.reshape(1, _H2), Wfc, bfc.reshape(1, _C),
    )


# speedup vs baseline: 9.6579x; 1.0007x over previous
import functools

import jax
import jax.numpy as jnp
from jax import lax
from jax.experimental import pallas as pl
from jax.experimental.pallas import tpu as pltpu
from jax.experimental.pallas import tpu_sc as plsc

_N = 10000
_E = 160000
_D = 256
_H1 = 512
_H2 = 128
_C = 4

_L = 128
_NACC = 10240
_RPT = _NACC // 16
_SENT = _N
_STG = 40
_EP = 163840
_NB1 = _EP // (16 * _L)
_NB2 = _EP // (32 * _L)


def _zero_rows(buf, ncols):
    z = jnp.zeros((16,), jnp.float32)

    def body(r, _):
        for k in range(ncols // 16):
            buf[r, pl.ds(k * 16, 16)] = z
        return 0

    lax.fori_loop(0, 128, body, 0)


def _fill_rows(buf, ncols, val):
    v = jnp.full((16,), val, jnp.float32)

    def body(r, _):
        for k in range(ncols // 16):
            buf[r, pl.ds(k * 16, 16)] = v
        return 0

    lax.fori_loop(0, 128, body, 0)


def _make_deg_kernel():
    mesh = plsc.VectorSubcoreMesh(core_axis_name="c", subcore_axis_name="s")

    @functools.partial(
        pl.kernel,
        out_type=jax.ShapeDtypeStruct((2 * _NACC, 128), jnp.float32),
        mesh=mesh,
        scratch_types=[
            pltpu.VMEM((_NB2, _L), jnp.int32),
            pltpu.VMEM((128, 128), jnp.float32),
            pltpu.VMEM_SHARED((_NACC, 128), jnp.float32),
        ],
    )
    def deg_kernel(dst_hbm, out_hbm, dst_v, gbuf, acc):
        c = lax.axis_index("c")
        s = lax.axis_index("s")
        w = c * 16 + s
        _zero_rows(gbuf, 128)
        base = s * _RPT
        for k in range(_RPT // 128):
            pltpu.sync_copy(gbuf, acc.at[pl.ds(base + k * 128, 128)])
        rem = _RPT % 128
        if rem:
            pltpu.sync_copy(gbuf.at[pl.ds(0, rem)],
                            acc.at[pl.ds(base + _RPT - rem, rem)])
        plsc.subcore_barrier()
        _fill_rows(gbuf, 128, 1.0)
        pltpu.sync_copy(dst_hbm.at[w], dst_v)

        def body(j, _):
            pltpu.sync_copy(gbuf, acc.at[dst_v.at[j]], add=True)
            return 0

        lax.fori_loop(0, _NB2, body, 0)
        plsc.subcore_barrier()
        pltpu.sync_copy(
            acc.at[pl.ds(base, _RPT)],
            out_hbm.at[pl.ds(c * _NACC + base, _RPT)],
        )

    return deg_kernel


def _make_scatter_kernel(nb):
    mesh = plsc.VectorSubcoreMesh(core_axis_name="c", subcore_axis_name="s")

    stage = _STG
    nstg = nb // stage

    def body_fn(table_hbm, src_hbm, dst_hbm, out_hbm, src_v, dst_v, buf_a,
                buf_b, gs_a, gs_b, ss_a, ss_b, acc):
        c = lax.axis_index("c")
        s = lax.axis_index("s")
        w = c * 16 + s
        _zero_rows(buf_a, 128)
        base = s * _RPT
        for k in range(_RPT // 128):
            pltpu.sync_copy(buf_a, acc.at[pl.ds(base + k * 128, 128)])
        rem = _RPT % 128
        if rem:
            pltpu.sync_copy(buf_a.at[pl.ds(0, rem)],
                            acc.at[pl.ds(base + _RPT - rem, rem)])
        plsc.subcore_barrier()

        bufs = [buf_a, buf_b]
        gsems = [gs_a, gs_b]
        ssems = [ss_a, ss_b]

        def waitg(k):
            pltpu.make_async_copy(
                table_hbm.at[pl.ds(0, _L)], bufs[k], gsems[k]).wait()

        def waits(k):
            pltpu.make_async_copy(
                bufs[k], acc.at[pl.ds(0, _L)], ssems[k]).wait()

        def gather(j, k):
            pltpu.async_copy(table_hbm.at[src_v.at[j]], bufs[k], gsems[k])

        def scatter(j, k):
            pltpu.async_copy(bufs[k], acc.at[dst_v.at[j]], ssems[k], add=True)

        def run_stage(st, _):
            pltpu.sync_copy(src_hbm.at[w, pl.ds(st * stage, stage)], src_v)
            pltpu.sync_copy(dst_hbm.at[w, pl.ds(st * stage, stage)], dst_v)
            gather(0, 0)

            def step(j, k, do_waits):
                waitg(k)
                scatter(j, k)
                kn = 1 - k
                if do_waits:
                    waits(kn)
                gather(j + 1, kn)

            def pipe(m, _):
                step(2 * m, 0, True)
                step(2 * m + 1, 1, True)
                return 0

            step(0, 0, False)
            step(1, 1, True)
            lax.fori_loop(1, stage // 2 - 1, pipe, 0)
            j = stage - 2
            waitg(0)
            scatter(j, 0)
            waits(1)
            gather(j + 1, 1)
            waitg(1)
            scatter(j + 1, 1)
            waits(0)
            waits(1)
            return 0

        lax.fori_loop(0, nstg, run_stage, 0)
        plsc.subcore_barrier()
        pltpu.sync_copy(
            acc.at[pl.ds(base, _RPT)],
            out_hbm.at[pl.ds(c * _NACC + base, _RPT)],
        )

    return functools.partial(
        pl.kernel,
        out_type=jax.ShapeDtypeStruct((2 * _NACC, 128), jnp.float32),
        mesh=mesh,
        scratch_types=[
            pltpu.VMEM((stage, _L), jnp.int32),
            pltpu.VMEM((stage, _L), jnp.int32),
            pltpu.VMEM((_L, 128), jnp.float32),
            pltpu.VMEM((_L, 128), jnp.float32),
            pltpu.SemaphoreType.DMA,
            pltpu.SemaphoreType.DMA,
            pltpu.SemaphoreType.DMA,
            pltpu.SemaphoreType.DMA,
            pltpu.VMEM_SHARED((_NACC, 128), jnp.float32),
        ],
    )(body_fn)


_BN = 1000


def _tc_prep(x, d0, d1):
    def body(x_ref, d0_ref, d1_ref, g_ref, dis_ref):
        deg = d0_ref[:, :1] + d1_ref[:, :1] + 1.0
        dis = lax.rsqrt(deg)
        dis_ref[...] = dis
        g_ref[...] = x_ref[...] * dis

    nblk = _N // _BN
    return pl.pallas_call(
        body,
        grid=(2, nblk),
        in_specs=[
            pl.BlockSpec((_BN, 128), lambda h, i: (i, h)),
            pl.BlockSpec((_BN, 128), lambda h, i: (i, 0)),
            pl.BlockSpec((_BN, 128), lambda h, i: (i, 0)),
        ],
        out_specs=[
            pl.BlockSpec((_BN, 128), lambda h, i: (h * nblk + i, 0)),
            pl.BlockSpec((_BN, 1), lambda h, i: (i, 0)),
        ],
        out_shape=[
            jax.ShapeDtypeStruct((2 * _N, 128), jnp.float32),
            jax.ShapeDtypeStruct((_N, 1), jnp.float32),
        ],
    )(x, d0, d1)


def _tc_mid(sa, sb, ga, gb, dis, w1a, w1b, b1, w2):
    def body(sa_ref, sb_ref, ga_ref, gb_ref, dis_ref, w1a_ref, w1b_ref,
             b1_ref, w2_ref, g2_ref):
        dis = dis_ref[...]
        aa = (sa_ref[...] + ga_ref[...]) * dis
        ab = (sb_ref[...] + gb_ref[...]) * dis
        h1 = jnp.dot(aa, w1a_ref[...], preferred_element_type=jnp.float32)
        h1 += jnp.dot(ab, w1b_ref[...], preferred_element_type=jnp.float32)
        h1 = jnp.maximum(h1 + b1_ref[...], 0.0)
        p2 = jnp.dot(h1, w2_ref[...], preferred_element_type=jnp.float32)
        g2_ref[...] = p2 * dis

    nblk = _N // _BN
    return pl.pallas_call(
        body,
        grid=(nblk,),
        in_specs=[
            pl.BlockSpec((_BN, 128), lambda i: (i, 0)),
            pl.BlockSpec((_BN, 128), lambda i: (i, 0)),
            pl.BlockSpec((_BN, 128), lambda i: (i, 0)),
            pl.BlockSpec((_BN, 128), lambda i: (i, 0)),
            pl.BlockSpec((_BN, 1), lambda i: (i, 0)),
            pl.BlockSpec((128, _H1), lambda i: (0, 0)),
            pl.BlockSpec((128, _H1), lambda i: (0, 0)),
            pl.BlockSpec((1, _H1), lambda i: (0, 0)),
            pl.BlockSpec((_H1, _H2), lambda i: (0, 0)),
        ],
        out_specs=pl.BlockSpec((_BN, _H2), lambda i: (i, 0)),
        out_shape=jax.ShapeDtypeStruct((_N, _H2), jnp.float32),
    )(sa, sb, ga, gb, dis, w1a, w1b, b1, w2)


def _tc_final(s2a, s2b, g2, dis, b2, wfc, bfc):
    def body(sa_ref, sb_ref, g2_ref, dis_ref, b2_ref, wfc_ref, bfc_ref,
             out_ref):
        a = (sa_ref[...] + sb_ref[...] + g2_ref[...]) * dis_ref[...]
        h2 = jnp.maximum(a + b2_ref[...], 0.0)
        logits = jnp.dot(h2, wfc_ref[...], preferred_element_type=jnp.float32)
        logits += bfc_ref[...]
        m = jnp.max(logits, axis=1, keepdims=True)
        z = logits - m
        lse = jnp.log(jnp.sum(jnp.exp(z), axis=1, keepdims=True))
        out_ref[...] = z - lse

    nblk = _N // _BN
    return pl.pallas_call(
        body,
        grid=(nblk,),
        in_specs=[
            pl.BlockSpec((_BN, _H2), lambda i: (i, 0)),
            pl.BlockSpec((_BN, _H2), lambda i: (i, 0)),
            pl.BlockSpec((_BN, _H2), lambda i: (i, 0)),
            pl.BlockSpec((_BN, 1), lambda i: (i, 0)),
            pl.BlockSpec((1, _H2), lambda i: (0, 0)),
            pl.BlockSpec((_H2, _C), lambda i: (0, 0)),
            pl.BlockSpec((1, _C), lambda i: (0, 0)),
        ],
        out_specs=pl.BlockSpec((_BN, _C), lambda i: (i, 0)),
        out_shape=jax.ShapeDtypeStruct((_N, _C), jnp.float32),
    )(s2a, s2b, g2, dis, b2, wfc, bfc)


@jax.jit
def kernel(x, edge_index, W1, b1, W2, b2, Wfc, bfc):
    src = edge_index[0]
    dst = edge_index[1]
    pad = _EP - _E
    srcp = jnp.concatenate([src, jnp.zeros((pad,), jnp.int32)])
    dstp = jnp.concatenate([dst, jnp.full((pad,), _SENT, jnp.int32)])

    src2 = srcp.reshape(32, _NB2, _L)
    dst2 = dstp.reshape(32, _NB2, _L)
    src1 = (srcp.reshape(1, 16, _NB1, _L)
            + jnp.array([0, _N], jnp.int32).reshape(2, 1, 1, 1)
            ).reshape(32, _NB1, _L)
    dst1 = jnp.broadcast_to(
        dstp.reshape(1, 16, _NB1, _L), (2, 16, _NB1, _L)
    ).reshape(32, _NB1, _L)

    degcat = _make_deg_kernel()(dstp.reshape(32, _NB2, _L))
    d0 = degcat[:_N]
    d1 = degcat[_NACC:_NACC + _N]

    gcat, dis = _tc_prep(x, d0, d1)

    s1cat = _make_scatter_kernel(_NB1)(gcat, src1, dst1)
    g2 = _tc_mid(
        s1cat[:_N], s1cat[_NACC:_NACC + _N],
        gcat[:_N], gcat[_N:],
        dis, W1[:128], W1[128:], b1.reshape(1, _H1), W2,
    )

    s2cat = _make_scatter_kernel(_NB2)(g2, src2, dst2)
    return _tc_final(
        s2cat[:_N], s2cat[_NACC:_NACC + _N], g2, dis,
        b2.reshape(1, _H2), Wfc, bfc.reshape(1, _C),
    )
